# baseline (speedup 1.0000x reference)
import functools

import jax
import jax.numpy as jnp
from jax import lax
from jax.experimental import pallas as pl
from jax.experimental.pallas import tpu as pltpu
from jax.experimental.pallas import tpu_sc as plsc

NC, NS, L = 2, 16, 16
NW = NC * NS
UNROLL = 8
RPS = 64
NBUF = 4
LOOKAHEAD = 2
RANGE = 4000


def _sc_body_a(M, B, nid_hbm, tall_hbm, tab_hbm,
               nid_v, table_v, tflat_v, negones_v, sem_nid, sem_out):
    wid = lax.axis_index("s") * NC + lax.axis_index("c")
    chunk = B // NW
    base = wid * chunk
    iota = lax.iota(jnp.int32, L)
    full = iota >= 0

    cp_nid = pltpu.async_copy(nid_hbm, nid_v, sem_nid)

    for k in range(8):
        negones_v[pl.ds(k * L, L)] = jnp.full((L,), -1, jnp.int32)
    rbase = wid * RANGE

    @pl.when(wid < M // RANGE)
    def _():
        def init(i, c):
            table_v[pl.ds(rbase + i * L, L)] = negones_v[pl.ds(0, L)]
            return c

        lax.fori_loop(0, RANGE // L, init, 0)

    cp_nid.wait()

    def body(j, c):
        idss = [nid_v[pl.ds((j * UNROLL + u) * L, L)] for u in range(UNROLL)]
        wins = [plsc.scan_count(ids)[1] for ids in idss]
        for u in range(UNROLL):
            pos = (j * UNROLL + u) * L + iota
            plsc.store_scatter(table_v, [idss[u]], pos, mask=wins[u])
        return c

    lax.fori_loop(0, B // (L * UNROLL), body, 0)

    for j in range(chunk // L):
        ids = nid_v[pl.ds(base + j * L, L)]
        t = plsc.load_gather(table_v, [ids], mask=full)
        tflat_v[pl.ds(j * L, L)] = t
    cp_tall = pltpu.async_copy(tflat_v, tall_hbm.at[pl.ds(base, chunk)],
                               sem_out)

    @pl.when(wid < M // RANGE)
    def _():
        pltpu.sync_copy(table_v.at[pl.ds(rbase, RANGE)],
                        tab_hbm.at[pl.ds(rbase, RANGE)])

    cp_tall.wait()


def _sc_body_b(M, B, D, nid_hbm, tall_hbm, msg_hbm, ts_hbm, memts_hbm,
               tab_hbm, newmem_hbm, newts_hbm, has_hbm,
               nid512_v, tflat_v, dst2d_v, ts_v, tabr_v, memtsr_v,
               hasb_v, ntsb_v, rb0, rb1, rb2, rb3,
               sg0, sg1, sg2, sg3, ss0, ss1, ss2, ss3, sem_misc):
    wid = lax.axis_index("s") * NC + lax.axis_index("c")
    chunk = B // NW
    base = wid * chunk
    iota = lax.iota(jnp.int32, L)
    full = iota >= 0
    rbase = wid * RANGE
    in_range = wid < M // RANGE

    cp_nid = pltpu.async_copy(nid_hbm.at[pl.ds(base, chunk)], nid512_v,
                              sem_misc)
    cp_t = pltpu.async_copy(tall_hbm.at[pl.ds(base, chunk)], tflat_v,
                            sem_misc)
    cp_ts = pltpu.async_copy(ts_hbm, ts_v, sem_misc)

    cp_nid.wait()
    cp_t.wait()
    for j in range(chunk // RPS):
        for u in range(RPS // L):
            dst2d_v[j, pl.ds(u * L, L)] = nid512_v[pl.ds(j * RPS + u * L, L)]

    n_it = chunk // RPS
    rbufs = (rb0, rb1, rb2, rb3)
    gsems = (sg0, sg1, sg2, sg3)
    ssems = (ss0, ss1, ss2, ss3)
    pend_g = [None] * NBUF
    pend_s = [None] * NBUF

    def gather(k):
        b = k % NBUF
        return pltpu.async_copy(msg_hbm.at[tflat_v.at[pl.ds(k * RPS, RPS)]],
                                rbufs[b], gsems[b])

    def scatter(k):
        b = k % NBUF
        return pltpu.async_copy(rbufs[b], newmem_hbm.at[dst2d_v.at[k]],
                                ssems[b])

    for k in range(min(LOOKAHEAD, n_it)):
        pend_g[k % NBUF] = gather(k)

    cp_ts.wait()

    @pl.when(in_range)
    def _():
        cp_a = pltpu.async_copy(tab_hbm.at[pl.ds(rbase, RANGE)], tabr_v,
                                sem_misc)
        cp_b = pltpu.async_copy(memts_hbm.at[pl.ds(rbase, RANGE)],
                                memtsr_v, sem_misc)
        cp_a.wait()
        cp_b.wait()

        def dense(i, c):
            tab = tabr_v[pl.ds(i * L, L)]
            has = tab >= 0
            tsg = plsc.load_gather(ts_v, [jnp.maximum(tab, 0)], mask=full)
            mts = memtsr_v[pl.ds(i * L, L)]
            hasb_v[pl.ds(i * L, L)] = jnp.where(has, 1, 0)
            ntsb_v[pl.ds(i * L, L)] = jnp.where(has, tsg, mts)
            return c

        lax.fori_loop(0, RANGE // L, dense, 0)
        pltpu.async_copy(hasb_v, has_hbm.at[pl.ds(rbase, RANGE)],
                         sem_misc).wait()
        pltpu.async_copy(ntsb_v, newts_hbm.at[pl.ds(rbase, RANGE)],
                         sem_misc).wait()

    for k in range(n_it):
        b = k % NBUF
        ka = k + LOOKAHEAD
        if ka < n_it:
            ba = ka % NBUF
            if pend_s[ba] is not None:
                pend_s[ba].wait()
                pend_s[ba] = None
            pend_g[ba] = gather(ka)
        pend_g[b].wait()
        pend_s[b] = scatter(k)
    for b in range(NBUF):
        if pend_s[b] is not None:
            pend_s[b].wait()


def _make_call_a(M, B, interpret=False):
    chunk = B // NW
    mesh = plsc.VectorSubcoreMesh(core_axis_name="c", subcore_axis_name="s",
                                  num_cores=NC, num_subcores=NS)
    return pl.kernel(
        functools.partial(_sc_body_a, M, B),
        out_type=(jax.ShapeDtypeStruct((B,), jnp.int32),
                  jax.ShapeDtypeStruct((M,), jnp.int32)),
        mesh=mesh,
        scratch_types=[
            pltpu.VMEM((B,), jnp.int32),
            pltpu.VMEM((M,), jnp.int32),
            pltpu.VMEM((chunk,), jnp.int32),
            pltpu.VMEM((128,), jnp.int32),
            pltpu.SemaphoreType.DMA,
            pltpu.SemaphoreType.DMA,
        ],
        interpret=interpret,
        compiler_params=pltpu.CompilerParams(needs_layout_passes=False),
        name="msg_agg_sc_table",
    )


def _make_call_b(M, B, D, interpret=False):
    chunk = B // NW
    mesh = plsc.VectorSubcoreMesh(core_axis_name="c", subcore_axis_name="s",
                                  num_cores=NC, num_subcores=NS)
    return pl.kernel(
        functools.partial(_sc_body_b, M, B, D),
        out_type=(jax.ShapeDtypeStruct((M,), jnp.float32),
                  jax.ShapeDtypeStruct((M,), jnp.int32)),
        mesh=mesh,
        scratch_types=[
            pltpu.VMEM((chunk,), jnp.int32),
            pltpu.VMEM((chunk,), jnp.int32),
            pltpu.VMEM((chunk // RPS, RPS), jnp.int32),
            pltpu.VMEM((B,), jnp.float32),
            pltpu.VMEM((RANGE,), jnp.int32),
            pltpu.VMEM((RANGE,), jnp.float32),
            pltpu.VMEM((RANGE,), jnp.int32),
            pltpu.VMEM((RANGE,), jnp.float32),
            pltpu.VMEM((RPS, D), jnp.float32),
            pltpu.VMEM((RPS, D), jnp.float32),
            pltpu.VMEM((RPS, D), jnp.float32),
            pltpu.VMEM((RPS, D), jnp.float32),
            pltpu.SemaphoreType.DMA,
            pltpu.SemaphoreType.DMA,
            pltpu.SemaphoreType.DMA,
            pltpu.SemaphoreType.DMA,
            pltpu.SemaphoreType.DMA,
            pltpu.SemaphoreType.DMA,
            pltpu.SemaphoreType.DMA,
            pltpu.SemaphoreType.DMA,
            pltpu.SemaphoreType.DMA,
        ],
        interpret=interpret,
        compiler_params=pltpu.CompilerParams(needs_layout_passes=False),
        name="msg_agg_sc_rows",
    )


def kernel(mem, mem_ts, node_ids, messages, timestamps):
    M, D = mem.shape
    B = node_ids.shape[0]
    newmem = jax.new_ref(mem)
    t_all, table = _make_call_a(M, B)(node_ids)
    new_ts, has = _make_call_b(M, B, D)(node_ids, t_all, messages,
                                        timestamps, mem_ts, table, newmem)
    return newmem[...], new_ts, has.astype(jnp.bool_)
